# baseline (speedup 1.0000x reference)
import functools
import math

import jax
import jax.numpy as jnp
import numpy as np
from jax import lax
from jax.experimental import pallas as pl
from jax.experimental.pallas import tpu as pltpu

_NUM_TOKENS = 4096
_NUM_EXPERTS = 16
_MODEL_DIM = 2048
_CAPACITY = max(math.ceil(_NUM_TOKENS / _NUM_EXPERTS * 1.0), 4)
_TB = 256
_ROWS = _TB * _NUM_EXPERTS
_XB = 512


def _threefry2x32(key0, key1, x0, x1):
    rotations = ((13, 15, 26, 6), (17, 29, 16, 24))

    def rol(x, d):
        return (x << np.uint32(d)) | (x >> np.uint32(32 - d))

    ks = (key0, key1, key0 ^ key1 ^ np.uint32(0x1BD11BDA))
    x0 = x0 + ks[0]
    x1 = x1 + ks[1]
    with np.errstate(over="ignore"):
        for i in range(5):
            for r in rotations[i % 2]:
                x0 = x0 + x1
                x1 = rol(x1, r)
                x1 = x1 ^ x0
            x0 = x0 + ks[(i + 1) % 3]
            x1 = x1 + ks[(i + 2) % 3] + np.uint32(i + 1)
    return x0, x1


@functools.lru_cache(maxsize=None)
def _rank_const() -> np.ndarray:
    n = _NUM_TOKENS * _NUM_EXPERTS
    idx = np.arange(n, dtype=np.uint32)
    b1, b2 = _threefry2x32(
        np.uint32(0), np.uint32(42), np.zeros(n, dtype=np.uint32), idx
    )
    bits = b1 ^ b2
    f = ((bits >> np.uint32(9)) | np.uint32(0x3F800000)).view(np.float32)
    r = np.maximum(np.float32(0.0), f - np.float32(1.0)).reshape(
        _NUM_TOKENS, _NUM_EXPERTS
    )
    order = np.argsort(-r, axis=0, kind="stable")
    rank = np.argsort(order, axis=0)
    return rank.astype(np.int32)


_RANK = _rank_const()

_E_REP = np.equal(
    np.arange(_ROWS, dtype=np.int32)[:, None] // _NUM_EXPERTS,
    np.arange(_TB, dtype=np.int32)[None, :],
).astype(np.float32)


def _tree_sum0(x):
    n = x.shape[0]
    chunks = [x[j : j + n // 8] for j in range(0, n, n // 8)]
    while len(chunks) > 1:
        chunks = [chunks[k] + chunks[k + 1] for k in range(0, len(chunks), 2)]
    return jnp.sum(chunks[0], axis=0, keepdims=True)


def _gates_body(x_ref, wg_ref, g_ref):
    logits = lax.dot_general(
        x_ref[...], wg_ref[...], (((1,), (1,)), ((), ())),
        preferred_element_type=jnp.float32,
    )
    m = jnp.max(logits, axis=1, keepdims=True)
    ex = jnp.exp(logits - m)
    g_ref[...] = ex / jnp.sum(ex, axis=1, keepdims=True)


def _fill_body(gates_ref, rank_ref, e_ref, comb_ref, disp_ref, laux_ref,
               cnt_ref, p_scr, lf_scr):
    i = pl.program_id(0)

    @pl.when(i == 0)
    def _routing():
        gates = gates_ref[...]
        m = jnp.max(gates, axis=1, keepdims=True)
        lane = lax.broadcasted_iota(jnp.int32, (_NUM_TOKENS, _NUM_EXPERTS), 1)
        es = jnp.min(
            jnp.where(gates == m, lane, _NUM_EXPERTS), axis=1, keepdims=True
        )
        mask1 = (lane == es).astype(jnp.int32)
        counts = _tree_sum0(mask1)
        me = _tree_sum0(gates) / _NUM_TOKENS
        ce = counts.astype(jnp.float32) / _NUM_TOKENS
        laux = jnp.sum(me * ce) * _NUM_EXPERTS

        rank = rank_ref[...]
        lo = jnp.zeros((1, _NUM_EXPERTS), jnp.int32)
        hi = jnp.full((1, _NUM_EXPERTS), _NUM_TOKENS + 1, jnp.int32)
        for _ in range(13):
            mid = (lo + hi) // 2
            cnt = _tree_sum0(jnp.where((mask1 == 1) & (rank < mid), 1, 0))
            ge = cnt >= _CAPACITY
            hi = jnp.where(ge, mid, hi)
            lo = jnp.where(ge, lo, mid)
        sel = mask1 * (rank < hi).astype(jnp.int32)

        csum = sel
        k = 1
        while k < _NUM_TOKENS:
            shifted = jnp.concatenate(
                [
                    jnp.zeros((k, _NUM_EXPERTS), jnp.int32),
                    csum[: _NUM_TOKENS - k, :],
                ],
                axis=0,
            )
            csum = csum + shifted
            k *= 2
        loc = jnp.sum((csum - 1) * sel, axis=1, keepdims=True)

        p_scr[...] = gates * sel.astype(jnp.float32)
        lf_scr[...] = jnp.broadcast_to(
            loc.astype(jnp.float32), (_NUM_TOKENS, _NUM_EXPERTS)
        )
        laux_ref[...] = jnp.full((8, _NUM_EXPERTS), laux, jnp.float32)
        cnt_ref[...] = jnp.broadcast_to(counts, (8, _NUM_EXPERTS))

    base = pl.multiple_of(i * _TB, _TB)
    psub = p_scr[pl.ds(base, _TB), :]
    lsub = lf_scr[pl.ds(base, _TB), :]
    zin = jnp.concatenate([psub, lsub], axis=1)
    z = lax.dot_general(
        e_ref[...], zin, (((1,), (0,)), ((), ())),
        precision=lax.Precision.HIGHEST,
        preferred_element_type=jnp.float32,
    )
    irow = lax.broadcasted_iota(jnp.int32, (_ROWS, _NUM_EXPERTS), 0)
    ie = lax.broadcasted_iota(jnp.int32, (_ROWS, _NUM_EXPERTS), 1)
    mm = ((irow % _NUM_EXPERTS) == ie).astype(jnp.float32)
    p2 = jnp.sum(z[:, :_NUM_EXPERTS] * mm, axis=1, keepdims=True)
    loc2 = z[:, _NUM_EXPERTS : _NUM_EXPERTS + 1].astype(jnp.int32)
    cc = lax.broadcasted_iota(jnp.int32, (_ROWS, _CAPACITY), 1)
    comb = jnp.where(cc == loc2, p2, 0.0).reshape(_TB, _NUM_EXPERTS, _CAPACITY)
    comb_ref[...] = comb
    disp_ref[...] = comb != 0.0


def kernel(input, wg_weight):
    gates = pl.pallas_call(
        _gates_body,
        grid=(_NUM_TOKENS // _XB,),
        in_specs=[
            pl.BlockSpec((_XB, _MODEL_DIM), lambda i: (i, 0)),
            pl.BlockSpec((_NUM_EXPERTS, _MODEL_DIM), lambda i: (0, 0)),
        ],
        out_specs=pl.BlockSpec((_XB, _NUM_EXPERTS), lambda i: (i, 0)),
        out_shape=jax.ShapeDtypeStruct((_NUM_TOKENS, _NUM_EXPERTS), jnp.float32),
    )(input, wg_weight)

    rank = jnp.asarray(_RANK)
    erep = jnp.asarray(_E_REP)
    comb, disp, laux, cnt = pl.pallas_call(
        _fill_body,
        grid=(_NUM_TOKENS // _TB,),
        in_specs=[
            pl.BlockSpec((_NUM_TOKENS, _NUM_EXPERTS), lambda i: (0, 0)),
            pl.BlockSpec((_NUM_TOKENS, _NUM_EXPERTS), lambda i: (0, 0)),
            pl.BlockSpec((_ROWS, _TB), lambda i: (0, 0)),
        ],
        out_specs=[
            pl.BlockSpec((_TB, _NUM_EXPERTS, _CAPACITY), lambda i: (i, 0, 0)),
            pl.BlockSpec((_TB, _NUM_EXPERTS, _CAPACITY), lambda i: (i, 0, 0)),
            pl.BlockSpec((8, _NUM_EXPERTS), lambda i: (0, 0)),
            pl.BlockSpec((8, _NUM_EXPERTS), lambda i: (0, 0)),
        ],
        out_shape=(
            jax.ShapeDtypeStruct(
                (_NUM_TOKENS, _NUM_EXPERTS, _CAPACITY), jnp.float32
            ),
            jax.ShapeDtypeStruct(
                (_NUM_TOKENS, _NUM_EXPERTS, _CAPACITY), jnp.bool_
            ),
            jax.ShapeDtypeStruct((8, _NUM_EXPERTS), jnp.float32),
            jax.ShapeDtypeStruct((8, _NUM_EXPERTS), jnp.int32),
        ),
        scratch_shapes=[
            pltpu.VMEM((_NUM_TOKENS, _NUM_EXPERTS), jnp.float32),
            pltpu.VMEM((_NUM_TOKENS, _NUM_EXPERTS), jnp.float32),
        ],
    )(gates, rank, erep)

    return (laux[0, 0], comb, disp, cnt[0])


# speedup vs baseline: 1.9292x; 1.9292x over previous
import functools
import math

import jax
import jax.numpy as jnp
import numpy as np
from jax import lax
from jax.experimental import pallas as pl

_NUM_TOKENS = 4096
_NUM_EXPERTS = 16
_MODEL_DIM = 2048
_CAPACITY = max(math.ceil(_NUM_TOKENS / _NUM_EXPERTS * 1.0), 4)
_TB = 512
_SUB = 32
_XB = 512


def _threefry2x32(key0, key1, x0, x1):
    rotations = ((13, 15, 26, 6), (17, 29, 16, 24))

    def rol(x, d):
        return (x << np.uint32(d)) | (x >> np.uint32(32 - d))

    ks = (key0, key1, key0 ^ key1 ^ np.uint32(0x1BD11BDA))
    x0 = x0 + ks[0]
    x1 = x1 + ks[1]
    with np.errstate(over="ignore"):
        for i in range(5):
            for r in rotations[i % 2]:
                x0 = x0 + x1
                x1 = rol(x1, r)
                x1 = x1 ^ x0
            x0 = x0 + ks[(i + 1) % 3]
            x1 = x1 + ks[(i + 2) % 3] + np.uint32(i + 1)
    return x0, x1


@functools.lru_cache(maxsize=None)
def _rank_const() -> np.ndarray:
    n = _NUM_TOKENS * _NUM_EXPERTS
    idx = np.arange(n, dtype=np.uint32)
    b1, b2 = _threefry2x32(
        np.uint32(0), np.uint32(42), np.zeros(n, dtype=np.uint32), idx
    )
    bits = b1 ^ b2
    f = ((bits >> np.uint32(9)) | np.uint32(0x3F800000)).view(np.float32)
    r = np.maximum(np.float32(0.0), f - np.float32(1.0)).reshape(
        _NUM_TOKENS, _NUM_EXPERTS
    )
    order = np.argsort(-r, axis=0, kind="stable")
    rank = np.argsort(order, axis=0)
    return rank.astype(np.int32)


_RANK = _rank_const()


def _tree_sum0(x):
    n = x.shape[0]
    chunks = [x[j : j + n // 8] for j in range(0, n, n // 8)]
    while len(chunks) > 1:
        chunks = [chunks[k] + chunks[k + 1] for k in range(0, len(chunks), 2)]
    return jnp.sum(chunks[0], axis=0, keepdims=True)


def _gates_body(x_ref, wg_ref, g_ref):
    logits = lax.dot_general(
        x_ref[...], wg_ref[...], (((1,), (1,)), ((), ())),
        preferred_element_type=jnp.float32,
    )
    m = jnp.max(logits, axis=1, keepdims=True)
    ex = jnp.exp(logits - m)
    g_ref[...] = ex / jnp.sum(ex, axis=1, keepdims=True)


def _select_body(gates_ref, rank_ref, p_ref, l_ref, laux_ref, cnt_ref):
    gates = gates_ref[...]
    m = jnp.max(gates, axis=1, keepdims=True)
    lane = lax.broadcasted_iota(jnp.int32, (_NUM_TOKENS, _NUM_EXPERTS), 1)
    es = jnp.min(
        jnp.where(gates == m, lane, _NUM_EXPERTS), axis=1, keepdims=True
    )
    mask1 = (lane == es).astype(jnp.int32)
    counts = _tree_sum0(mask1)
    me = _tree_sum0(gates) / _NUM_TOKENS
    ce = counts.astype(jnp.float32) / _NUM_TOKENS
    laux = jnp.sum(me * ce) * _NUM_EXPERTS

    rank = rank_ref[...]
    lo = jnp.zeros((1, _NUM_EXPERTS), jnp.int32)
    hi = jnp.full((1, _NUM_EXPERTS), _NUM_TOKENS + 1, jnp.int32)
    for _ in range(13):
        mid = (lo + hi) // 2
        cnt = _tree_sum0(jnp.where((mask1 == 1) & (rank < mid), 1, 0))
        ge = cnt >= _CAPACITY
        hi = jnp.where(ge, mid, hi)
        lo = jnp.where(ge, lo, mid)
    sel = mask1 * (rank < hi).astype(jnp.int32)

    csum = sel
    k = 1
    while k < _NUM_TOKENS:
        shifted = jnp.concatenate(
            [jnp.zeros((k, _NUM_EXPERTS), jnp.int32), csum[: _NUM_TOKENS - k, :]],
            axis=0,
        )
        csum = csum + shifted
        k *= 2
    loc = jnp.sum((csum - 1) * sel, axis=1, keepdims=True)

    p_ref[...] = gates * sel.astype(jnp.float32)
    iota_c = lax.broadcasted_iota(jnp.int32, (_NUM_TOKENS, _CAPACITY), 1)
    l_ref[...] = (iota_c == loc).astype(jnp.bfloat16)
    laux_ref[...] = jnp.full((8, _NUM_EXPERTS), laux, jnp.float32)
    cnt_ref[...] = jnp.broadcast_to(counts, (8, _NUM_EXPERTS))


def _fill_body(p_ref, l_ref, comb_ref, disp_ref):
    p = p_ref[...]
    lh = l_ref[...]
    rows = _SUB * _NUM_EXPERTS
    ir = lax.broadcasted_iota(jnp.int32, (rows, _SUB), 0)
    it = lax.broadcasted_iota(jnp.int32, (rows, _SUB), 1)
    ef = ((ir // _NUM_EXPERTS) == it).astype(jnp.float32)
    eb = ef.astype(jnp.bfloat16)
    irow = lax.broadcasted_iota(jnp.int32, (rows, _NUM_EXPERTS), 0)
    ie = lax.broadcasted_iota(jnp.int32, (rows, _NUM_EXPERTS), 1)
    mm = ((irow % _NUM_EXPERTS) == ie).astype(jnp.float32)
    for t in range(_TB // _SUB):
        lsub = lh[t * _SUB : (t + 1) * _SUB, :]
        psub = p[t * _SUB : (t + 1) * _SUB, :]
        el = lax.dot_general(
            eb, lsub, (((1,), (0,)), ((), ())), preferred_element_type=jnp.float32
        )
        q = lax.dot_general(
            ef, psub, (((1,), (0,)), ((), ())),
            precision=lax.Precision.HIGHEST,
            preferred_element_type=jnp.float32,
        )
        p2 = jnp.sum(q * mm, axis=1, keepdims=True)
        comb = (el * p2).reshape(_SUB, _NUM_EXPERTS, _CAPACITY)
        comb_ref[t * _SUB : (t + 1) * _SUB, :, :] = comb
        disp_ref[t * _SUB : (t + 1) * _SUB, :, :] = comb != 0.0


def kernel(input, wg_weight):
    gates = pl.pallas_call(
        _gates_body,
        grid=(_NUM_TOKENS // _XB,),
        in_specs=[
            pl.BlockSpec((_XB, _MODEL_DIM), lambda i: (i, 0)),
            pl.BlockSpec((_NUM_EXPERTS, _MODEL_DIM), lambda i: (0, 0)),
        ],
        out_specs=pl.BlockSpec((_XB, _NUM_EXPERTS), lambda i: (i, 0)),
        out_shape=jax.ShapeDtypeStruct((_NUM_TOKENS, _NUM_EXPERTS), jnp.float32),
    )(input, wg_weight)

    rank = jnp.asarray(_RANK)
    p, l, laux, cnt = pl.pallas_call(
        _select_body,
        out_shape=(
            jax.ShapeDtypeStruct((_NUM_TOKENS, _NUM_EXPERTS), jnp.float32),
            jax.ShapeDtypeStruct((_NUM_TOKENS, _CAPACITY), jnp.bfloat16),
            jax.ShapeDtypeStruct((8, _NUM_EXPERTS), jnp.float32),
            jax.ShapeDtypeStruct((8, _NUM_EXPERTS), jnp.int32),
        ),
    )(gates, rank)

    comb, disp = pl.pallas_call(
        _fill_body,
        grid=(_NUM_TOKENS // _TB,),
        in_specs=[
            pl.BlockSpec((_TB, _NUM_EXPERTS), lambda i: (i, 0)),
            pl.BlockSpec((_TB, _CAPACITY), lambda i: (i, 0)),
        ],
        out_specs=[
            pl.BlockSpec((_TB, _NUM_EXPERTS, _CAPACITY), lambda i: (i, 0, 0)),
            pl.BlockSpec((_TB, _NUM_EXPERTS, _CAPACITY), lambda i: (i, 0, 0)),
        ],
        out_shape=(
            jax.ShapeDtypeStruct(
                (_NUM_TOKENS, _NUM_EXPERTS, _CAPACITY), jnp.float32
            ),
            jax.ShapeDtypeStruct(
                (_NUM_TOKENS, _NUM_EXPERTS, _CAPACITY), jnp.bool_
            ),
        ),
    )(p, l)

    return (laux[0, 0], comb, disp, cnt[0])


# speedup vs baseline: 2.0496x; 1.0624x over previous
import functools
import math

import jax
import jax.numpy as jnp
import numpy as np
from jax import lax
from jax.experimental import pallas as pl

_NUM_TOKENS = 4096
_NUM_EXPERTS = 16
_MODEL_DIM = 2048
_CAPACITY = max(math.ceil(_NUM_TOKENS / _NUM_EXPERTS * 1.0), 4)
_TB = 512
_SUB = 32
_XB = 512


def _threefry2x32(key0, key1, x0, x1):
    rotations = ((13, 15, 26, 6), (17, 29, 16, 24))

    def rol(x, d):
        return (x << np.uint32(d)) | (x >> np.uint32(32 - d))

    ks = (key0, key1, key0 ^ key1 ^ np.uint32(0x1BD11BDA))
    x0 = x0 + ks[0]
    x1 = x1 + ks[1]
    with np.errstate(over="ignore"):
        for i in range(5):
            for r in rotations[i % 2]:
                x0 = x0 + x1
                x1 = rol(x1, r)
                x1 = x1 ^ x0
            x0 = x0 + ks[(i + 1) % 3]
            x1 = x1 + ks[(i + 2) % 3] + np.uint32(i + 1)
    return x0, x1


@functools.lru_cache(maxsize=None)
def _rank_const() -> np.ndarray:
    n = _NUM_TOKENS * _NUM_EXPERTS
    idx = np.arange(n, dtype=np.uint32)
    b1, b2 = _threefry2x32(
        np.uint32(0), np.uint32(42), np.zeros(n, dtype=np.uint32), idx
    )
    bits = b1 ^ b2
    f = ((bits >> np.uint32(9)) | np.uint32(0x3F800000)).view(np.float32)
    r = np.maximum(np.float32(0.0), f - np.float32(1.0)).reshape(
        _NUM_TOKENS, _NUM_EXPERTS
    )
    order = np.argsort(-r, axis=0, kind="stable")
    rank = np.argsort(order, axis=0)
    return rank.astype(np.int32)


_RANK = _rank_const()


def _tree_sum0(x):
    n = x.shape[0]
    chunks = [x[j : j + n // 8] for j in range(0, n, n // 8)]
    while len(chunks) > 1:
        chunks = [chunks[k] + chunks[k + 1] for k in range(0, len(chunks), 2)]
    return jnp.sum(chunks[0], axis=0, keepdims=True)


def _gates_body(x_ref, wg_ref, g_ref):
    logits = lax.dot_general(
        x_ref[...], wg_ref[...], (((1,), (1,)), ((), ())),
        preferred_element_type=jnp.float32,
    )
    m = jnp.max(logits, axis=1, keepdims=True)
    ex = jnp.exp(logits - m)
    g_ref[...] = ex / jnp.sum(ex, axis=1, keepdims=True)


def _select_body(gates_ref, rank_ref, p_ref, l_ref, laux_ref, cnt_ref):
    gates = gates_ref[...]
    m = jnp.max(gates, axis=1, keepdims=True)
    lane = lax.broadcasted_iota(jnp.int32, (_NUM_TOKENS, _NUM_EXPERTS), 1)
    es = jnp.min(
        jnp.where(gates == m, lane, _NUM_EXPERTS), axis=1, keepdims=True
    )
    mask1 = (lane == es).astype(jnp.int32)
    counts = _tree_sum0(mask1)
    me = _tree_sum0(gates) / _NUM_TOKENS
    ce = counts.astype(jnp.float32) / _NUM_TOKENS
    laux = jnp.sum(me * ce) * _NUM_EXPERTS

    rank = rank_ref[...]
    lo = jnp.zeros((1, _NUM_EXPERTS), jnp.int32)
    hi = jnp.full((1, _NUM_EXPERTS), _NUM_TOKENS + 1, jnp.int32)
    for _ in range(13):
        mid = (lo + hi) // 2
        cnt = _tree_sum0(jnp.where((mask1 == 1) & (rank < mid), 1, 0))
        ge = cnt >= _CAPACITY
        hi = jnp.where(ge, mid, hi)
        lo = jnp.where(ge, lo, mid)
    sel = mask1 * (rank < hi).astype(jnp.int32)

    csum = sel
    k = 1
    while k < _NUM_TOKENS:
        shifted = jnp.concatenate(
            [jnp.zeros((k, _NUM_EXPERTS), jnp.int32), csum[: _NUM_TOKENS - k, :]],
            axis=0,
        )
        csum = csum + shifted
        k *= 2
    loc = jnp.sum((csum - 1) * sel, axis=1, keepdims=True)

    p_ref[...] = (gates * sel.astype(jnp.float32)).astype(jnp.bfloat16)
    iota_c = lax.broadcasted_iota(jnp.int32, (_NUM_TOKENS, _CAPACITY), 1)
    l_ref[...] = (iota_c == loc).astype(jnp.bfloat16)
    laux_ref[...] = jnp.full((8, _NUM_EXPERTS), laux, jnp.float32)
    cnt_ref[...] = jnp.broadcast_to(counts, (8, _NUM_EXPERTS))


def _fill_body(p_ref, l_ref, comb_ref, disp_ref):
    p = p_ref[...]
    lh = l_ref[...]
    rows = _SUB * _NUM_EXPERTS
    ir = lax.broadcasted_iota(jnp.int32, (rows, _SUB), 0)
    it = lax.broadcasted_iota(jnp.int32, (rows, _SUB), 1)
    eb = ((ir // _NUM_EXPERTS) == it).astype(jnp.bfloat16)
    irow = lax.broadcasted_iota(jnp.int32, (rows, _NUM_EXPERTS), 0)
    ie = lax.broadcasted_iota(jnp.int32, (rows, _NUM_EXPERTS), 1)
    mm = ((irow % _NUM_EXPERTS) == ie).astype(jnp.float32)
    for t in range(_TB // _SUB):
        lsub = lh[t * _SUB : (t + 1) * _SUB, :]
        psub = p[t * _SUB : (t + 1) * _SUB, :]
        el = lax.dot_general(
            eb, lsub, (((1,), (0,)), ((), ())), preferred_element_type=jnp.float32
        )
        q = lax.dot_general(
            eb, psub, (((1,), (0,)), ((), ())), preferred_element_type=jnp.float32
        )
        p2 = jnp.sum(q * mm, axis=1, keepdims=True)
        comb = (el * p2).reshape(_SUB, _NUM_EXPERTS, _CAPACITY)
        comb_ref[t * _SUB : (t + 1) * _SUB, :, :] = comb
        disp_ref[t * _SUB : (t + 1) * _SUB, :, :] = comb != 0.0


def kernel(input, wg_weight):
    gates = pl.pallas_call(
        _gates_body,
        grid=(_NUM_TOKENS // _XB,),
        in_specs=[
            pl.BlockSpec((_XB, _MODEL_DIM), lambda i: (i, 0)),
            pl.BlockSpec((_NUM_EXPERTS, _MODEL_DIM), lambda i: (0, 0)),
        ],
        out_specs=pl.BlockSpec((_XB, _NUM_EXPERTS), lambda i: (i, 0)),
        out_shape=jax.ShapeDtypeStruct((_NUM_TOKENS, _NUM_EXPERTS), jnp.float32),
    )(input, wg_weight)

    rank = jnp.asarray(_RANK)
    p, l, laux, cnt = pl.pallas_call(
        _select_body,
        out_shape=(
            jax.ShapeDtypeStruct((_NUM_TOKENS, _NUM_EXPERTS), jnp.bfloat16),
            jax.ShapeDtypeStruct((_NUM_TOKENS, _CAPACITY), jnp.bfloat16),
            jax.ShapeDtypeStruct((8, _NUM_EXPERTS), jnp.float32),
            jax.ShapeDtypeStruct((8, _NUM_EXPERTS), jnp.int32),
        ),
    )(gates, rank)

    comb, disp = pl.pallas_call(
        _fill_body,
        grid=(_NUM_TOKENS // _TB,),
        in_specs=[
            pl.BlockSpec((_TB, _NUM_EXPERTS), lambda i: (i, 0)),
            pl.BlockSpec((_TB, _CAPACITY), lambda i: (i, 0)),
        ],
        out_specs=[
            pl.BlockSpec((_TB, _NUM_EXPERTS, _CAPACITY), lambda i: (i, 0, 0)),
            pl.BlockSpec((_TB, _NUM_EXPERTS, _CAPACITY), lambda i: (i, 0, 0)),
        ],
        out_shape=(
            jax.ShapeDtypeStruct(
                (_NUM_TOKENS, _NUM_EXPERTS, _CAPACITY), jnp.float32
            ),
            jax.ShapeDtypeStruct(
                (_NUM_TOKENS, _NUM_EXPERTS, _CAPACITY), jnp.bool_
            ),
        ),
    )(p, l)

    return (laux[0, 0], comb, disp, cnt[0])


# speedup vs baseline: 2.0692x; 1.0096x over previous
import functools
import math

import jax
import jax.numpy as jnp
import numpy as np
from jax import lax
from jax.experimental import pallas as pl

_NUM_TOKENS = 4096
_NUM_EXPERTS = 16
_MODEL_DIM = 2048
_CAPACITY = max(math.ceil(_NUM_TOKENS / _NUM_EXPERTS * 1.0), 4)
_TB = 256
_SUB = 32
_XB = 512


def _threefry2x32(key0, key1, x0, x1):
    rotations = ((13, 15, 26, 6), (17, 29, 16, 24))

    def rol(x, d):
        return (x << np.uint32(d)) | (x >> np.uint32(32 - d))

    ks = (key0, key1, key0 ^ key1 ^ np.uint32(0x1BD11BDA))
    x0 = x0 + ks[0]
    x1 = x1 + ks[1]
    with np.errstate(over="ignore"):
        for i in range(5):
            for r in rotations[i % 2]:
                x0 = x0 + x1
                x1 = rol(x1, r)
                x1 = x1 ^ x0
            x0 = x0 + ks[(i + 1) % 3]
            x1 = x1 + ks[(i + 2) % 3] + np.uint32(i + 1)
    return x0, x1


@functools.lru_cache(maxsize=None)
def _rank_const() -> np.ndarray:
    n = _NUM_TOKENS * _NUM_EXPERTS
    idx = np.arange(n, dtype=np.uint32)
    b1, b2 = _threefry2x32(
        np.uint32(0), np.uint32(42), np.zeros(n, dtype=np.uint32), idx
    )
    bits = b1 ^ b2
    f = ((bits >> np.uint32(9)) | np.uint32(0x3F800000)).view(np.float32)
    r = np.maximum(np.float32(0.0), f - np.float32(1.0)).reshape(
        _NUM_TOKENS, _NUM_EXPERTS
    )
    order = np.argsort(-r, axis=0, kind="stable")
    rank = np.argsort(order, axis=0)
    return rank.astype(np.int32)


_RANK = _rank_const()


def _tree_sum0(x):
    n = x.shape[0]
    chunks = [x[j : j + n // 8] for j in range(0, n, n // 8)]
    while len(chunks) > 1:
        chunks = [chunks[k] + chunks[k + 1] for k in range(0, len(chunks), 2)]
    return jnp.sum(chunks[0], axis=0, keepdims=True)


def _gates_body(x_ref, wg_ref, g_ref):
    logits = lax.dot_general(
        x_ref[...], wg_ref[...], (((1,), (1,)), ((), ())),
        preferred_element_type=jnp.float32,
    )
    m = jnp.max(logits, axis=1, keepdims=True)
    ex = jnp.exp(logits - m)
    g_ref[...] = ex / jnp.sum(ex, axis=1, keepdims=True)


def _select_body(gates_ref, rank_ref, p_ref, l_ref, laux_ref, cnt_ref):
    gates = gates_ref[...]
    m = jnp.max(gates, axis=1, keepdims=True)
    lane = lax.broadcasted_iota(jnp.int32, (_NUM_TOKENS, _NUM_EXPERTS), 1)
    es = jnp.min(
        jnp.where(gates == m, lane, _NUM_EXPERTS), axis=1, keepdims=True
    )
    mask1 = (lane == es).astype(jnp.int32)
    counts = _tree_sum0(mask1)
    me = _tree_sum0(gates) / _NUM_TOKENS
    ce = counts.astype(jnp.float32) / _NUM_TOKENS
    laux = jnp.sum(me * ce) * _NUM_EXPERTS

    rank = rank_ref[...]
    lo = jnp.zeros((1, _NUM_EXPERTS), jnp.int32)
    hi = jnp.full((1, _NUM_EXPERTS), _NUM_TOKENS + 1, jnp.int32)
    for _ in range(13):
        mid = (lo + hi) // 2
        cnt = _tree_sum0(jnp.where((mask1 == 1) & (rank < mid), 1, 0))
        ge = cnt >= _CAPACITY
        hi = jnp.where(ge, mid, hi)
        lo = jnp.where(ge, lo, mid)
    sel = mask1 * (rank < hi).astype(jnp.int32)

    csum = sel
    k = 1
    while k < _NUM_TOKENS:
        shifted = jnp.concatenate(
            [jnp.zeros((k, _NUM_EXPERTS), jnp.int32), csum[: _NUM_TOKENS - k, :]],
            axis=0,
        )
        csum = csum + shifted
        k *= 2
    loc = jnp.sum((csum - 1) * sel, axis=1, keepdims=True)

    p_ref[...] = (gates * sel.astype(jnp.float32)).astype(jnp.bfloat16)
    iota_c = lax.broadcasted_iota(jnp.int32, (_NUM_TOKENS, _CAPACITY), 1)
    l_ref[...] = (iota_c == loc).astype(jnp.bfloat16)
    laux_ref[...] = jnp.full((8, _NUM_EXPERTS), laux, jnp.float32)
    cnt_ref[...] = jnp.broadcast_to(counts, (8, _NUM_EXPERTS))


def _fill_body(p_ref, l_ref, comb_ref, disp_ref):
    p = p_ref[...]
    lh = l_ref[...]
    rows = _SUB * _NUM_EXPERTS
    ir = lax.broadcasted_iota(jnp.int32, (rows, _SUB), 0)
    it = lax.broadcasted_iota(jnp.int32, (rows, _SUB), 1)
    eb = ((ir // _NUM_EXPERTS) == it).astype(jnp.bfloat16)
    irow = lax.broadcasted_iota(jnp.int32, (rows, _NUM_EXPERTS), 0)
    ie = lax.broadcasted_iota(jnp.int32, (rows, _NUM_EXPERTS), 1)
    mm = ((irow % _NUM_EXPERTS) == ie).astype(jnp.float32)
    for t in range(_TB // _SUB):
        lsub = lh[t * _SUB : (t + 1) * _SUB, :]
        psub = p[t * _SUB : (t + 1) * _SUB, :]
        el = lax.dot_general(
            eb, lsub, (((1,), (0,)), ((), ())), preferred_element_type=jnp.float32
        )
        q = lax.dot_general(
            eb, psub, (((1,), (0,)), ((), ())), preferred_element_type=jnp.float32
        )
        p2 = jnp.sum(q * mm, axis=1, keepdims=True)
        comb = (el * p2).reshape(_SUB, _NUM_EXPERTS, _CAPACITY)
        comb_ref[t * _SUB : (t + 1) * _SUB, :, :] = comb
        disp_ref[t * _SUB : (t + 1) * _SUB, :, :] = comb != 0.0


def kernel(input, wg_weight):
    gates = pl.pallas_call(
        _gates_body,
        grid=(_NUM_TOKENS // _XB,),
        in_specs=[
            pl.BlockSpec((_XB, _MODEL_DIM), lambda i: (i, 0)),
            pl.BlockSpec((_NUM_EXPERTS, _MODEL_DIM), lambda i: (0, 0)),
        ],
        out_specs=pl.BlockSpec((_XB, _NUM_EXPERTS), lambda i: (i, 0)),
        out_shape=jax.ShapeDtypeStruct((_NUM_TOKENS, _NUM_EXPERTS), jnp.float32),
    )(input, wg_weight)

    rank = jnp.asarray(_RANK)
    p, l, laux, cnt = pl.pallas_call(
        _select_body,
        out_shape=(
            jax.ShapeDtypeStruct((_NUM_TOKENS, _NUM_EXPERTS), jnp.bfloat16),
            jax.ShapeDtypeStruct((_NUM_TOKENS, _CAPACITY), jnp.bfloat16),
            jax.ShapeDtypeStruct((8, _NUM_EXPERTS), jnp.float32),
            jax.ShapeDtypeStruct((8, _NUM_EXPERTS), jnp.int32),
        ),
    )(gates, rank)

    comb, disp = pl.pallas_call(
        _fill_body,
        grid=(_NUM_TOKENS // _TB,),
        in_specs=[
            pl.BlockSpec((_TB, _NUM_EXPERTS), lambda i: (i, 0)),
            pl.BlockSpec((_TB, _CAPACITY), lambda i: (i, 0)),
        ],
        out_specs=[
            pl.BlockSpec((_TB, _NUM_EXPERTS, _CAPACITY), lambda i: (i, 0, 0)),
            pl.BlockSpec((_TB, _NUM_EXPERTS, _CAPACITY), lambda i: (i, 0, 0)),
        ],
        out_shape=(
            jax.ShapeDtypeStruct(
                (_NUM_TOKENS, _NUM_EXPERTS, _CAPACITY), jnp.float32
            ),
            jax.ShapeDtypeStruct(
                (_NUM_TOKENS, _NUM_EXPERTS, _CAPACITY), jnp.bool_
            ),
        ),
    )(p, l)

    return (laux[0, 0], comb, disp, cnt[0])


# speedup vs baseline: 2.0842x; 1.0072x over previous
import functools
import math

import jax
import jax.numpy as jnp
import numpy as np
from jax import lax
from jax.experimental import pallas as pl

_NUM_TOKENS = 4096
_NUM_EXPERTS = 16
_MODEL_DIM = 2048
_CAPACITY = max(math.ceil(_NUM_TOKENS / _NUM_EXPERTS * 1.0), 4)
_TB = 256
_SUB = 64
_XB = 512


def _threefry2x32(key0, key1, x0, x1):
    rotations = ((13, 15, 26, 6), (17, 29, 16, 24))

    def rol(x, d):
        return (x << np.uint32(d)) | (x >> np.uint32(32 - d))

    ks = (key0, key1, key0 ^ key1 ^ np.uint32(0x1BD11BDA))
    x0 = x0 + ks[0]
    x1 = x1 + ks[1]
    with np.errstate(over="ignore"):
        for i in range(5):
            for r in rotations[i % 2]:
                x0 = x0 + x1
                x1 = rol(x1, r)
                x1 = x1 ^ x0
            x0 = x0 + ks[(i + 1) % 3]
            x1 = x1 + ks[(i + 2) % 3] + np.uint32(i + 1)
    return x0, x1


@functools.lru_cache(maxsize=None)
def _rank_const() -> np.ndarray:
    n = _NUM_TOKENS * _NUM_EXPERTS
    idx = np.arange(n, dtype=np.uint32)
    b1, b2 = _threefry2x32(
        np.uint32(0), np.uint32(42), np.zeros(n, dtype=np.uint32), idx
    )
    bits = b1 ^ b2
    f = ((bits >> np.uint32(9)) | np.uint32(0x3F800000)).view(np.float32)
    r = np.maximum(np.float32(0.0), f - np.float32(1.0)).reshape(
        _NUM_TOKENS, _NUM_EXPERTS
    )
    order = np.argsort(-r, axis=0, kind="stable")
    rank = np.argsort(order, axis=0)
    return rank.astype(np.int32)


_RANK = _rank_const()


def _tree_sum0(x):
    n = x.shape[0]
    chunks = [x[j : j + n // 8] for j in range(0, n, n // 8)]
    while len(chunks) > 1:
        chunks = [chunks[k] + chunks[k + 1] for k in range(0, len(chunks), 2)]
    return jnp.sum(chunks[0], axis=0, keepdims=True)


def _gates_body(x_ref, wg_ref, g_ref):
    logits = lax.dot_general(
        x_ref[...], wg_ref[...], (((1,), (1,)), ((), ())),
        preferred_element_type=jnp.float32,
    )
    m = jnp.max(logits, axis=1, keepdims=True)
    ex = jnp.exp(logits - m)
    g_ref[...] = ex / jnp.sum(ex, axis=1, keepdims=True)


def _select_body(gates_ref, rank_ref, p_ref, l_ref, laux_ref, cnt_ref):
    gates = gates_ref[...]
    m = jnp.max(gates, axis=1, keepdims=True)
    lane = lax.broadcasted_iota(jnp.int32, (_NUM_TOKENS, _NUM_EXPERTS), 1)
    es = jnp.min(
        jnp.where(gates == m, lane, _NUM_EXPERTS), axis=1, keepdims=True
    )
    mask1 = (lane == es).astype(jnp.int32)
    counts = _tree_sum0(mask1)
    me = _tree_sum0(gates) / _NUM_TOKENS
    ce = counts.astype(jnp.float32) / _NUM_TOKENS
    laux = jnp.sum(me * ce) * _NUM_EXPERTS

    rank = rank_ref[...]
    lo = jnp.zeros((1, _NUM_EXPERTS), jnp.int32)
    hi = jnp.full((1, _NUM_EXPERTS), _NUM_TOKENS + 1, jnp.int32)
    for _ in range(13):
        mid = (lo + hi) // 2
        cnt = _tree_sum0(jnp.where((mask1 == 1) & (rank < mid), 1, 0))
        ge = cnt >= _CAPACITY
        hi = jnp.where(ge, mid, hi)
        lo = jnp.where(ge, lo, mid)
    sel = mask1 * (rank < hi).astype(jnp.int32)

    csum = sel
    k = 1
    while k < _NUM_TOKENS:
        shifted = jnp.concatenate(
            [jnp.zeros((k, _NUM_EXPERTS), jnp.int32), csum[: _NUM_TOKENS - k, :]],
            axis=0,
        )
        csum = csum + shifted
        k *= 2
    loc = jnp.sum((csum - 1) * sel, axis=1, keepdims=True)

    p_ref[...] = (gates * sel.astype(jnp.float32)).astype(jnp.bfloat16)
    iota_c = lax.broadcasted_iota(jnp.int32, (_NUM_TOKENS, _CAPACITY), 1)
    l_ref[...] = (iota_c == loc).astype(jnp.bfloat16)
    laux_ref[...] = jnp.full((8, _NUM_EXPERTS), laux, jnp.float32)
    cnt_ref[...] = jnp.broadcast_to(counts, (8, _NUM_EXPERTS))


def _fill_body(p_ref, l_ref, comb_ref, disp_ref):
    p = p_ref[...]
    lh = l_ref[...]
    rows = _SUB * _NUM_EXPERTS
    ir = lax.broadcasted_iota(jnp.int32, (rows, _SUB), 0)
    it = lax.broadcasted_iota(jnp.int32, (rows, _SUB), 1)
    eb = ((ir // _NUM_EXPERTS) == it).astype(jnp.bfloat16)
    irow = lax.broadcasted_iota(jnp.int32, (rows, _NUM_EXPERTS), 0)
    ie = lax.broadcasted_iota(jnp.int32, (rows, _NUM_EXPERTS), 1)
    mm = ((irow % _NUM_EXPERTS) == ie).astype(jnp.float32)
    for t in range(_TB // _SUB):
        lsub = lh[t * _SUB : (t + 1) * _SUB, :]
        psub = p[t * _SUB : (t + 1) * _SUB, :]
        el = lax.dot_general(
            eb, lsub, (((1,), (0,)), ((), ())), preferred_element_type=jnp.float32
        )
        q = lax.dot_general(
            eb, psub, (((1,), (0,)), ((), ())), preferred_element_type=jnp.float32
        )
        p2 = jnp.sum(q * mm, axis=1, keepdims=True)
        comb = (el * p2).reshape(_SUB, _NUM_EXPERTS, _CAPACITY)
        comb_ref[t * _SUB : (t + 1) * _SUB, :, :] = comb
        disp_ref[t * _SUB : (t + 1) * _SUB, :, :] = comb != 0.0


def kernel(input, wg_weight):
    gates = pl.pallas_call(
        _gates_body,
        grid=(_NUM_TOKENS // _XB,),
        in_specs=[
            pl.BlockSpec((_XB, _MODEL_DIM), lambda i: (i, 0)),
            pl.BlockSpec((_NUM_EXPERTS, _MODEL_DIM), lambda i: (0, 0)),
        ],
        out_specs=pl.BlockSpec((_XB, _NUM_EXPERTS), lambda i: (i, 0)),
        out_shape=jax.ShapeDtypeStruct((_NUM_TOKENS, _NUM_EXPERTS), jnp.float32),
    )(input, wg_weight)

    rank = jnp.asarray(_RANK)
    p, l, laux, cnt = pl.pallas_call(
        _select_body,
        out_shape=(
            jax.ShapeDtypeStruct((_NUM_TOKENS, _NUM_EXPERTS), jnp.bfloat16),
            jax.ShapeDtypeStruct((_NUM_TOKENS, _CAPACITY), jnp.bfloat16),
            jax.ShapeDtypeStruct((8, _NUM_EXPERTS), jnp.float32),
            jax.ShapeDtypeStruct((8, _NUM_EXPERTS), jnp.int32),
        ),
    )(gates, rank)

    comb, disp = pl.pallas_call(
        _fill_body,
        grid=(_NUM_TOKENS // _TB,),
        in_specs=[
            pl.BlockSpec((_TB, _NUM_EXPERTS), lambda i: (i, 0)),
            pl.BlockSpec((_TB, _CAPACITY), lambda i: (i, 0)),
        ],
        out_specs=[
            pl.BlockSpec((_TB, _NUM_EXPERTS, _CAPACITY), lambda i: (i, 0, 0)),
            pl.BlockSpec((_TB, _NUM_EXPERTS, _CAPACITY), lambda i: (i, 0, 0)),
        ],
        out_shape=(
            jax.ShapeDtypeStruct(
                (_NUM_TOKENS, _NUM_EXPERTS, _CAPACITY), jnp.float32
            ),
            jax.ShapeDtypeStruct(
                (_NUM_TOKENS, _NUM_EXPERTS, _CAPACITY), jnp.bool_
            ),
        ),
    )(p, l)

    return (laux[0, 0], comb, disp, cnt[0])


# speedup vs baseline: 2.1465x; 1.0299x over previous
import functools
import math

import jax
import jax.numpy as jnp
import numpy as np
from jax import lax
from jax.experimental import pallas as pl
from jax.experimental.pallas import tpu as pltpu

_NUM_TOKENS = 4096
_NUM_EXPERTS = 16
_MODEL_DIM = 2048
_CAPACITY = max(math.ceil(_NUM_TOKENS / _NUM_EXPERTS * 1.0), 4)
_TB = 256
_SUB = 64
_XB = 512


def _threefry2x32(key0, key1, x0, x1):
    rotations = ((13, 15, 26, 6), (17, 29, 16, 24))

    def rol(x, d):
        return (x << np.uint32(d)) | (x >> np.uint32(32 - d))

    ks = (key0, key1, key0 ^ key1 ^ np.uint32(0x1BD11BDA))
    x0 = x0 + ks[0]
    x1 = x1 + ks[1]
    with np.errstate(over="ignore"):
        for i in range(5):
            for r in rotations[i % 2]:
                x0 = x0 + x1
                x1 = rol(x1, r)
                x1 = x1 ^ x0
            x0 = x0 + ks[(i + 1) % 3]
            x1 = x1 + ks[(i + 2) % 3] + np.uint32(i + 1)
    return x0, x1


@functools.lru_cache(maxsize=None)
def _rank_const() -> np.ndarray:
    n = _NUM_TOKENS * _NUM_EXPERTS
    idx = np.arange(n, dtype=np.uint32)
    b1, b2 = _threefry2x32(
        np.uint32(0), np.uint32(42), np.zeros(n, dtype=np.uint32), idx
    )
    bits = b1 ^ b2
    f = ((bits >> np.uint32(9)) | np.uint32(0x3F800000)).view(np.float32)
    r = np.maximum(np.float32(0.0), f - np.float32(1.0)).reshape(
        _NUM_TOKENS, _NUM_EXPERTS
    )
    order = np.argsort(-r, axis=0, kind="stable")
    rank = np.argsort(order, axis=0)
    return rank.astype(np.int32)


_RANK = _rank_const()


def _tree_sum0(x):
    n = x.shape[0]
    chunks = [x[j : j + n // 8] for j in range(0, n, n // 8)]
    while len(chunks) > 1:
        chunks = [chunks[k] + chunks[k + 1] for k in range(0, len(chunks), 2)]
    return jnp.sum(chunks[0], axis=0, keepdims=True)


def _gates_body(x_ref, wg_ref, g_ref):
    logits = lax.dot_general(
        x_ref[...], wg_ref[...], (((1,), (1,)), ((), ())),
        preferred_element_type=jnp.float32,
    )
    m = jnp.max(logits, axis=1, keepdims=True)
    ex = jnp.exp(logits - m)
    g_ref[...] = ex / jnp.sum(ex, axis=1, keepdims=True)


def _select_body(gates_ref, rank_ref, p_ref, l_ref, laux_ref, cnt_ref):
    _select_core(gates_ref, rank_ref, p_ref, l_ref, laux_ref, cnt_ref)


def _select_core(gates_ref, rank_ref, p_ref, l_ref, laux_ref, cnt_ref):
    gates = gates_ref[...]
    m = jnp.max(gates, axis=1, keepdims=True)
    lane = lax.broadcasted_iota(jnp.int32, (_NUM_TOKENS, _NUM_EXPERTS), 1)
    es = jnp.min(
        jnp.where(gates == m, lane, _NUM_EXPERTS), axis=1, keepdims=True
    )
    mask1 = (lane == es).astype(jnp.int32)
    counts = _tree_sum0(mask1)
    me = _tree_sum0(gates) / _NUM_TOKENS
    ce = counts.astype(jnp.float32) / _NUM_TOKENS
    laux = jnp.sum(me * ce) * _NUM_EXPERTS

    rank = rank_ref[...]
    lo = jnp.zeros((1, _NUM_EXPERTS), jnp.int32)
    hi = jnp.full((1, _NUM_EXPERTS), _NUM_TOKENS + 1, jnp.int32)
    for _ in range(13):
        mid = (lo + hi) // 2
        cnt = _tree_sum0(jnp.where((mask1 == 1) & (rank < mid), 1, 0))
        ge = cnt >= _CAPACITY
        hi = jnp.where(ge, mid, hi)
        lo = jnp.where(ge, lo, mid)
    sel = mask1 * (rank < hi).astype(jnp.int32)

    csum = sel
    k = 1
    while k < _NUM_TOKENS:
        shifted = jnp.concatenate(
            [jnp.zeros((k, _NUM_EXPERTS), jnp.int32), csum[: _NUM_TOKENS - k, :]],
            axis=0,
        )
        csum = csum + shifted
        k *= 2
    loc = jnp.sum((csum - 1) * sel, axis=1, keepdims=True)

    p_ref[...] = (gates * sel.astype(jnp.float32)).astype(jnp.bfloat16)
    iota_c = lax.broadcasted_iota(jnp.int32, (_NUM_TOKENS, _CAPACITY), 1)
    l_ref[...] = (iota_c == loc).astype(jnp.bfloat16)
    laux_ref[...] = jnp.full((8, _NUM_EXPERTS), laux, jnp.float32)
    cnt_ref[...] = jnp.broadcast_to(counts, (8, _NUM_EXPERTS))


def _fused_fill_body(gates_ref, rank_ref, comb_ref, disp_ref, laux_ref,
                     cnt_ref, p_scr, l_scr):
    i = pl.program_id(0)

    @pl.when(i == 0)
    def _routing():
        _select_core(gates_ref, rank_ref, p_scr, l_scr, laux_ref, cnt_ref)

    base = pl.multiple_of(i * _TB, _TB)
    p = p_scr[pl.ds(base, _TB), :]
    lh = l_scr[pl.ds(base, _TB), :]
    _fill_block(p, lh, comb_ref, disp_ref)


def _fill_body(p_ref, l_ref, comb_ref, disp_ref):
    p = p_ref[...]
    lh = l_ref[...]
    _fill_block(p, lh, comb_ref, disp_ref)


def _fill_block(p, lh, comb_ref, disp_ref):
    rows = _SUB * _NUM_EXPERTS
    ir = lax.broadcasted_iota(jnp.int32, (rows, _SUB), 0)
    it = lax.broadcasted_iota(jnp.int32, (rows, _SUB), 1)
    eb = ((ir // _NUM_EXPERTS) == it).astype(jnp.bfloat16)
    irow = lax.broadcasted_iota(jnp.int32, (rows, _NUM_EXPERTS), 0)
    ie = lax.broadcasted_iota(jnp.int32, (rows, _NUM_EXPERTS), 1)
    mm = ((irow % _NUM_EXPERTS) == ie).astype(jnp.float32)
    for t in range(_TB // _SUB):
        lsub = lh[t * _SUB : (t + 1) * _SUB, :]
        psub = p[t * _SUB : (t + 1) * _SUB, :]
        el = lax.dot_general(
            eb, lsub, (((1,), (0,)), ((), ())), preferred_element_type=jnp.float32
        )
        q = lax.dot_general(
            eb, psub, (((1,), (0,)), ((), ())), preferred_element_type=jnp.float32
        )
        p2 = jnp.sum(q * mm, axis=1, keepdims=True)
        comb = (el * p2).reshape(_SUB, _NUM_EXPERTS, _CAPACITY)
        comb_ref[t * _SUB : (t + 1) * _SUB, :, :] = comb
        disp_ref[t * _SUB : (t + 1) * _SUB, :, :] = comb != 0.0


def kernel(input, wg_weight):
    gates = pl.pallas_call(
        _gates_body,
        grid=(_NUM_TOKENS // _XB,),
        in_specs=[
            pl.BlockSpec((_XB, _MODEL_DIM), lambda i: (i, 0)),
            pl.BlockSpec((_NUM_EXPERTS, _MODEL_DIM), lambda i: (0, 0)),
        ],
        out_specs=pl.BlockSpec((_XB, _NUM_EXPERTS), lambda i: (i, 0)),
        out_shape=jax.ShapeDtypeStruct((_NUM_TOKENS, _NUM_EXPERTS), jnp.float32),
    )(input, wg_weight)

    rank = jnp.asarray(_RANK)
    comb, disp, laux, cnt = pl.pallas_call(
        _fused_fill_body,
        grid=(_NUM_TOKENS // _TB,),
        in_specs=[
            pl.BlockSpec((_NUM_TOKENS, _NUM_EXPERTS), lambda i: (0, 0)),
            pl.BlockSpec((_NUM_TOKENS, _NUM_EXPERTS), lambda i: (0, 0)),
        ],
        out_specs=[
            pl.BlockSpec((_TB, _NUM_EXPERTS, _CAPACITY), lambda i: (i, 0, 0)),
            pl.BlockSpec((_TB, _NUM_EXPERTS, _CAPACITY), lambda i: (i, 0, 0)),
            pl.BlockSpec((8, _NUM_EXPERTS), lambda i: (0, 0)),
            pl.BlockSpec((8, _NUM_EXPERTS), lambda i: (0, 0)),
        ],
        out_shape=(
            jax.ShapeDtypeStruct(
                (_NUM_TOKENS, _NUM_EXPERTS, _CAPACITY), jnp.float32
            ),
            jax.ShapeDtypeStruct(
                (_NUM_TOKENS, _NUM_EXPERTS, _CAPACITY), jnp.bool_
            ),
            jax.ShapeDtypeStruct((8, _NUM_EXPERTS), jnp.float32),
            jax.ShapeDtypeStruct((8, _NUM_EXPERTS), jnp.int32),
        ),
        scratch_shapes=[
            pltpu.VMEM((_NUM_TOKENS, _NUM_EXPERTS), jnp.bfloat16),
            pltpu.VMEM((_NUM_TOKENS, _CAPACITY), jnp.bfloat16),
        ],
    )(gates, rank)

    return (laux[0, 0], comb, disp, cnt[0])


# speedup vs baseline: 2.1800x; 1.0156x over previous
import functools
import math

import jax
import jax.numpy as jnp
import numpy as np
from jax import lax
from jax.experimental import pallas as pl
from jax.experimental.pallas import tpu as pltpu

_NUM_TOKENS = 4096
_NUM_EXPERTS = 16
_MODEL_DIM = 2048
_CAPACITY = max(math.ceil(_NUM_TOKENS / _NUM_EXPERTS * 1.0), 4)
_TB = 256
_SUB = 64
_XB = 1024


def _threefry2x32(key0, key1, x0, x1):
    rotations = ((13, 15, 26, 6), (17, 29, 16, 24))

    def rol(x, d):
        return (x << np.uint32(d)) | (x >> np.uint32(32 - d))

    ks = (key0, key1, key0 ^ key1 ^ np.uint32(0x1BD11BDA))
    x0 = x0 + ks[0]
    x1 = x1 + ks[1]
    with np.errstate(over="ignore"):
        for i in range(5):
            for r in rotations[i % 2]:
                x0 = x0 + x1
                x1 = rol(x1, r)
                x1 = x1 ^ x0
            x0 = x0 + ks[(i + 1) % 3]
            x1 = x1 + ks[(i + 2) % 3] + np.uint32(i + 1)
    return x0, x1


@functools.lru_cache(maxsize=None)
def _rank_const() -> np.ndarray:
    n = _NUM_TOKENS * _NUM_EXPERTS
    idx = np.arange(n, dtype=np.uint32)
    b1, b2 = _threefry2x32(
        np.uint32(0), np.uint32(42), np.zeros(n, dtype=np.uint32), idx
    )
    bits = b1 ^ b2
    f = ((bits >> np.uint32(9)) | np.uint32(0x3F800000)).view(np.float32)
    r = np.maximum(np.float32(0.0), f - np.float32(1.0)).reshape(
        _NUM_TOKENS, _NUM_EXPERTS
    )
    order = np.argsort(-r, axis=0, kind="stable")
    rank = np.argsort(order, axis=0)
    return rank.astype(np.int32)


_RANK = _rank_const()


def _tree_sum0(x):
    n = x.shape[0]
    chunks = [x[j : j + n // 8] for j in range(0, n, n // 8)]
    while len(chunks) > 1:
        chunks = [chunks[k] + chunks[k + 1] for k in range(0, len(chunks), 2)]
    return jnp.sum(chunks[0], axis=0, keepdims=True)


def _gates_body(x_ref, wg_ref, g_ref):
    logits = lax.dot_general(
        x_ref[...], wg_ref[...], (((1,), (1,)), ((), ())),
        preferred_element_type=jnp.float32,
    )
    m = jnp.max(logits, axis=1, keepdims=True)
    ex = jnp.exp(logits - m)
    g_ref[...] = ex / jnp.sum(ex, axis=1, keepdims=True)


def _select_body(gates_ref, rank_ref, p_ref, l_ref, laux_ref, cnt_ref):
    _select_core(gates_ref, rank_ref, p_ref, l_ref, laux_ref, cnt_ref)


def _select_core(gates_ref, rank_ref, p_ref, l_ref, laux_ref, cnt_ref):
    gates = gates_ref[...]
    m = jnp.max(gates, axis=1, keepdims=True)
    lane = lax.broadcasted_iota(jnp.int32, (_NUM_TOKENS, _NUM_EXPERTS), 1)
    es = jnp.min(
        jnp.where(gates == m, lane, _NUM_EXPERTS), axis=1, keepdims=True
    )
    mask1 = (lane == es).astype(jnp.int32)
    counts = _tree_sum0(mask1)
    me = _tree_sum0(gates) / _NUM_TOKENS
    ce = counts.astype(jnp.float32) / _NUM_TOKENS
    laux = jnp.sum(me * ce) * _NUM_EXPERTS

    rank = rank_ref[...]
    lo = jnp.zeros((1, _NUM_EXPERTS), jnp.int32)
    hi = jnp.full((1, _NUM_EXPERTS), _NUM_TOKENS + 1, jnp.int32)
    for _ in range(13):
        mid = (lo + hi) // 2
        cnt = _tree_sum0(jnp.where((mask1 == 1) & (rank < mid), 1, 0))
        ge = cnt >= _CAPACITY
        hi = jnp.where(ge, mid, hi)
        lo = jnp.where(ge, lo, mid)
    sel = mask1 * (rank < hi).astype(jnp.int32)

    csum = sel
    k = 1
    while k < _NUM_TOKENS:
        shifted = jnp.concatenate(
            [jnp.zeros((k, _NUM_EXPERTS), jnp.int32), csum[: _NUM_TOKENS - k, :]],
            axis=0,
        )
        csum = csum + shifted
        k *= 2
    loc = jnp.sum((csum - 1) * sel, axis=1, keepdims=True)

    p_ref[...] = (gates * sel.astype(jnp.float32)).astype(jnp.bfloat16)
    iota_c = lax.broadcasted_iota(jnp.int32, (_NUM_TOKENS, _CAPACITY), 1)
    l_ref[...] = (iota_c == loc).astype(jnp.bfloat16)
    laux_ref[...] = jnp.full((8, _NUM_EXPERTS), laux, jnp.float32)
    cnt_ref[...] = jnp.broadcast_to(counts, (8, _NUM_EXPERTS))


def _fused_fill_body(gates_ref, rank_ref, comb_ref, disp_ref, laux_ref,
                     cnt_ref, p_scr, l_scr):
    i = pl.program_id(0)

    @pl.when(i == 0)
    def _routing():
        _select_core(gates_ref, rank_ref, p_scr, l_scr, laux_ref, cnt_ref)

    base = pl.multiple_of(i * _TB, _TB)
    p = p_scr[pl.ds(base, _TB), :]
    lh = l_scr[pl.ds(base, _TB), :]
    _fill_block(p, lh, comb_ref, disp_ref)


def _fill_body(p_ref, l_ref, comb_ref, disp_ref):
    p = p_ref[...]
    lh = l_ref[...]
    _fill_block(p, lh, comb_ref, disp_ref)


def _fill_block(p, lh, comb_ref, disp_ref):
    rows = _SUB * _NUM_EXPERTS
    ir = lax.broadcasted_iota(jnp.int32, (rows, _SUB), 0)
    it = lax.broadcasted_iota(jnp.int32, (rows, _SUB), 1)
    eb = ((ir // _NUM_EXPERTS) == it).astype(jnp.bfloat16)
    irow = lax.broadcasted_iota(jnp.int32, (rows, _NUM_EXPERTS), 0)
    ie = lax.broadcasted_iota(jnp.int32, (rows, _NUM_EXPERTS), 1)
    mm = ((irow % _NUM_EXPERTS) == ie).astype(jnp.float32)
    for t in range(_TB // _SUB):
        lsub = lh[t * _SUB : (t + 1) * _SUB, :]
        psub = p[t * _SUB : (t + 1) * _SUB, :]
        el = lax.dot_general(
            eb, lsub, (((1,), (0,)), ((), ())), preferred_element_type=jnp.float32
        )
        q = lax.dot_general(
            eb, psub, (((1,), (0,)), ((), ())), preferred_element_type=jnp.float32
        )
        p2 = jnp.sum(q * mm, axis=1, keepdims=True)
        comb = (el * p2).reshape(_SUB, _NUM_EXPERTS, _CAPACITY)
        comb_ref[t * _SUB : (t + 1) * _SUB, :, :] = comb
        disp_ref[t * _SUB : (t + 1) * _SUB, :, :] = comb != 0.0


def kernel(input, wg_weight):
    gates = pl.pallas_call(
        _gates_body,
        grid=(_NUM_TOKENS // _XB,),
        in_specs=[
            pl.BlockSpec((_XB, _MODEL_DIM), lambda i: (i, 0)),
            pl.BlockSpec((_NUM_EXPERTS, _MODEL_DIM), lambda i: (0, 0)),
        ],
        out_specs=pl.BlockSpec((_XB, _NUM_EXPERTS), lambda i: (i, 0)),
        out_shape=jax.ShapeDtypeStruct((_NUM_TOKENS, _NUM_EXPERTS), jnp.float32),
    )(input, wg_weight)

    rank = jnp.asarray(_RANK)
    comb, disp, laux, cnt = pl.pallas_call(
        _fused_fill_body,
        grid=(_NUM_TOKENS // _TB,),
        in_specs=[
            pl.BlockSpec((_NUM_TOKENS, _NUM_EXPERTS), lambda i: (0, 0)),
            pl.BlockSpec((_NUM_TOKENS, _NUM_EXPERTS), lambda i: (0, 0)),
        ],
        out_specs=[
            pl.BlockSpec((_TB, _NUM_EXPERTS, _CAPACITY), lambda i: (i, 0, 0)),
            pl.BlockSpec((_TB, _NUM_EXPERTS, _CAPACITY), lambda i: (i, 0, 0)),
            pl.BlockSpec((8, _NUM_EXPERTS), lambda i: (0, 0)),
            pl.BlockSpec((8, _NUM_EXPERTS), lambda i: (0, 0)),
        ],
        out_shape=(
            jax.ShapeDtypeStruct(
                (_NUM_TOKENS, _NUM_EXPERTS, _CAPACITY), jnp.float32
            ),
            jax.ShapeDtypeStruct(
                (_NUM_TOKENS, _NUM_EXPERTS, _CAPACITY), jnp.bool_
            ),
            jax.ShapeDtypeStruct((8, _NUM_EXPERTS), jnp.float32),
            jax.ShapeDtypeStruct((8, _NUM_EXPERTS), jnp.int32),
        ),
        scratch_shapes=[
            pltpu.VMEM((_NUM_TOKENS, _NUM_EXPERTS), jnp.bfloat16),
            pltpu.VMEM((_NUM_TOKENS, _CAPACITY), jnp.bfloat16),
        ],
    )(gates, rank)

    return (laux[0, 0], comb, disp, cnt[0])
